# baseline (speedup 1.0000x reference)
import jax
import jax.numpy as jnp
from jax import lax
from jax.experimental import pallas as pl
from jax.experimental.pallas import tpu as pltpu
from jax.experimental.pallas import tpu_sc as plsc

NUM_CONT = 13
NUM_CAT = 26
NCOL = NUM_CONT + NUM_CAT
VOCAB1 = 100001
VOCAB1P = 100008
TW = 128
D = 16
B = 16384
OUT_W = NUM_CONT + NUM_CAT * D
OUT_WP = 512

NC = 2
NS = 16
NW = NC * NS
BPW = B // NW
SB = 16
NSUB = BPW // SB
XROW = B * NCOL // 128
XPW = BPW * NCOL // 128
XST = 160


def _body(x_hbm, tab_a_hbm, tab_b_hbm, out_hbm, xf_v, idx_v, rows_v, out_v,
          sem):
  wid = lax.axis_index("s") * NC + lax.axis_index("c")
  base = wid * BPW

  ofs = (wid * XPW) % 8
  pltpu.sync_copy(x_hbm.at[pl.ds(wid * XPW - ofs, XST)], xf_v)
  eoff = ofs * 128

  lanes = lax.iota(jnp.int32, 16)
  for f in range(NUM_CAT):

    @pl.loop(0, BPW // 16)
    def _transp(k):
      p = (k * 16 + lanes) * NCOL + (NUM_CONT + f) + eoff
      idx_v[f, pl.ds(k * 16, 16)] = plsc.load_gather(xf_v, [p >> 7, p & 127])

  @pl.loop(0, NSUB)
  def _round(t):
    row0 = base + t * SB

    @pl.loop(0, NUM_CAT // 2)
    def _fire_a(f):
      pltpu.make_async_copy(
          tab_a_hbm.at[f].at[idx_v.at[f, pl.ds(t * SB, SB)]],
          rows_v.at[pl.ds(f * SB, SB)],
          sem,
      ).start()

    @pl.loop(NUM_CAT // 2, NUM_CAT)
    def _fire_b(f):
      pltpu.make_async_copy(
          tab_b_hbm.at[f - NUM_CAT // 2].at[idx_v.at[f, pl.ds(t * SB, SB)]],
          rows_v.at[pl.ds(f * SB, SB)],
          sem,
      ).start()

    @pl.loop(0, SB)
    def _cont(j):
      p = (t * SB + j) * NCOL + lanes + eoff
      vals = plsc.load_gather(xf_v, [p >> 7, p & 127])
      out_v[j, pl.ds(0, D)] = vals.astype(jnp.float32)

    @pl.loop(0, NUM_CAT)
    def _drain(f):
      pltpu.make_async_copy(
          tab_a_hbm.at[0].at[idx_v.at[0, pl.ds(0, SB)]],
          rows_v.at[pl.ds(0, SB)],
          sem,
      ).wait()

    for f in range(NUM_CAT):

      @pl.loop(0, SB)
      def _repack(j):
        out_v[j, pl.ds(NUM_CONT + D * f, D)] = rows_v[f * SB + j, pl.ds(0, D)]

    pltpu.sync_copy(out_v, out_hbm.at[pl.ds(row0, SB)])


@jax.jit
def _run(xfull, tab_a, tab_b):
  mesh = plsc.VectorSubcoreMesh(
      core_axis_name="c", subcore_axis_name="s", num_cores=NC)
  f = pl.kernel(
      _body,
      out_type=jax.ShapeDtypeStruct((B, OUT_WP), jnp.float32),
      mesh=mesh,
      compiler_params=pltpu.CompilerParams(
          use_tc_tiling_on_sc=False, needs_layout_passes=False),
      scratch_types=[
          pltpu.VMEM((XST, 128), jnp.int32),
          pltpu.VMEM((NUM_CAT, BPW), jnp.int32),
          pltpu.VMEM((NUM_CAT * SB, TW), jnp.float32),
          pltpu.VMEM((SB, OUT_WP), jnp.float32),
          pltpu.SemaphoreType.DMA,
      ],
  )
  return f(xfull, tab_a, tab_b)


def kernel(input_x, tables):
  xfull = input_x.astype(jnp.int32).reshape(XROW, 128)
  tab_a = jnp.pad(
      tables[:NUM_CAT // 2], ((0, 0), (0, VOCAB1P - VOCAB1), (0, TW - D)))
  tab_b = jnp.pad(
      tables[NUM_CAT // 2:], ((0, 0), (0, VOCAB1P - VOCAB1), (0, TW - D)))
  return _run(xfull, tab_a, tab_b)[:, :OUT_W]


# speedup vs baseline: 1.4474x; 1.4474x over previous
import jax
import jax.numpy as jnp
from jax import lax
from jax.experimental import pallas as pl
from jax.experimental.pallas import tpu as pltpu
from jax.experimental.pallas import tpu_sc as plsc

NUM_CONT = 13
NUM_CAT = 26
NCOL = NUM_CONT + NUM_CAT
VOCAB1 = 100001
VOCAB1P = 100008
TW = 128
D = 16
B = 16384
OUT_W = NUM_CONT + NUM_CAT * D
OUT_WP = 512

NC = 2
NS = 16
NW = NC * NS
BPW = B // NW
SB = 16
NSUB = BPW // SB
XROW = B * NCOL // 128
XPW = BPW * NCOL // 128
XST = 160


def _body(x_hbm, tab_hbm, out_hbm, xf_v, idx_v, rows_v, out_v, sem):
  wid = lax.axis_index("s") * NC + lax.axis_index("c")
  base = wid * BPW

  ofs = (wid * XPW) % 8
  pltpu.sync_copy(x_hbm.at[pl.ds(wid * XPW - ofs, XST)], xf_v)
  eoff = ofs * 128

  lanes = lax.iota(jnp.int32, 16)
  for f in range(NUM_CAT):

    @pl.loop(0, BPW // 16)
    def _transp(k):
      p = (k * 16 + lanes) * NCOL + (NUM_CONT + f) + eoff
      idx_v[f, pl.ds(k * 16, 16)] = plsc.load_gather(xf_v, [p >> 7, p & 127])

  @pl.loop(0, NSUB)
  def _round(t):
    row0 = base + t * SB

    @pl.loop(0, NUM_CAT)
    def _fire(f):
      pltpu.make_async_copy(
          tab_hbm.at[f].at[idx_v.at[f, pl.ds(t * SB, SB)]],
          rows_v.at[pl.ds(f * SB, SB)],
          sem,
      ).start()

    @pl.loop(0, SB)
    def _cont(j):
      p = (t * SB + j) * NCOL + lanes + eoff
      vals = plsc.load_gather(xf_v, [p >> 7, p & 127])
      out_v[j, pl.ds(0, D)] = vals.astype(jnp.float32)

    @pl.loop(0, NUM_CAT)
    def _drain(f):
      pltpu.make_async_copy(
          tab_hbm.at[0].at[idx_v.at[0, pl.ds(0, SB)]],
          rows_v.at[pl.ds(0, SB)],
          sem,
      ).wait()

    for f in range(NUM_CAT):

      @pl.loop(0, SB)
      def _repack(j):
        out_v[j, pl.ds(NUM_CONT + D * f, D)] = rows_v[f * SB + j, pl.ds(0, D)]

    pltpu.sync_copy(out_v, out_hbm.at[pl.ds(row0, SB)])


@jax.jit
def _run(xfull, tab):
  mesh = plsc.VectorSubcoreMesh(
      core_axis_name="c", subcore_axis_name="s", num_cores=NC)
  f = pl.kernel(
      _body,
      out_type=jax.ShapeDtypeStruct((B, OUT_WP), jnp.float32),
      mesh=mesh,
      compiler_params=pltpu.CompilerParams(
          use_tc_tiling_on_sc=False, needs_layout_passes=False),
      scratch_types=[
          pltpu.VMEM((XST, 128), jnp.int32),
          pltpu.VMEM((NUM_CAT, BPW), jnp.int32),
          pltpu.VMEM((NUM_CAT * SB, TW), jnp.float32),
          pltpu.VMEM((SB, OUT_WP), jnp.float32),
          pltpu.SemaphoreType.DMA,
      ],
  )
  return f(xfull, tab)


def kernel(input_x, tables):
  xfull = input_x.astype(jnp.int32).reshape(XROW, 128)
  tabp = jnp.pad(tables, ((0, 0), (0, VOCAB1P - VOCAB1), (0, TW - D)))
  return _run(xfull, tabp)[:, :OUT_W]
